# Initial kernel scaffold; baseline (speedup 1.0000x reference)
import jax
import jax.numpy as jnp
from jax.experimental import pallas as pl


def kernel(z_e, dictionary, beta):
    raise NotImplementedError("write your pallas kernel here")


# speedup vs baseline: 1.6818x; 1.6818x over previous
import jax
import jax.numpy as jnp
from jax.experimental import pallas as pl

NUM_EMBEDDINGS = 1024
EMBEDDING_DIM = 64
COMMITMENT_COST = 0.25
SPARSITY_LEVEL = 4
BATCH = 8192
BLOCK = 256
HI = jax.lax.Precision.HIGHEST


def _prep_kernel(dn_ref, dnt_ref, g_ref):
    dn = dn_ref[...]
    dnt = dnt_ref[...]
    g_ref[...] = jax.lax.dot_general(
        dnt.astype(jnp.bfloat16), dn.astype(jnp.bfloat16),
        (((1,), (0,)), ((), ())),
        preferred_element_type=jnp.float32)


def _omp_kernel(zf_ref, dn_ref, dnt_ref, g_ref,
                gamma_ref, recon_ref, sse_ref, ent_ref):
    f32 = jnp.float32
    K = NUM_EMBEDDINGS
    zb = zf_ref[...]
    dn = dn_ref[...]
    dnt = dnt_ref[...]
    G = g_ref[...]

    hbar = jax.lax.dot_general(
        dnt.astype(jnp.bfloat16), zb.astype(jnp.bfloat16),
        (((1,), (0,)), ((), ())), preferred_element_type=jnp.float32)
    h = hbar
    iota = jax.lax.broadcasted_iota(jnp.int32, (K, BLOCK), 0)
    avail = jnp.ones((K, BLOCK), f32)

    ohs = []
    grows = []
    hbs = []
    g0 = None
    L = {}

    for step in range(SPARSITY_LEVEL):
        k = step + 1
        absh = jnp.abs(h) * avail
        m = jnp.max(absh, axis=0, keepdims=True)
        cand = jnp.where(absh >= m, iota, K)
        idx = jnp.min(cand, axis=0, keepdims=True)
        oh = (iota == idx).astype(f32)
        avail = avail * (1.0 - oh)
        gj = jax.lax.dot_general(G, oh, (((1,), (0,)), ((), ())),
                                 precision=HI)
        hbj = jnp.sum(oh * hbar, axis=0, keepdims=True)
        ohs.append(oh)
        hbs.append(hbj)
        if k < SPARSITY_LEVEL:
            if k == 1:
                g0 = gj
            gbf = jax.lax.dot_general(
                G.astype(jnp.bfloat16), oh.astype(jnp.bfloat16),
                (((1,), (0,)), ((), ())),
                preferred_element_type=jnp.float32)
            grows.append(gbf)

        if k == 1:
            L[(1, 1)] = jnp.ones((1, BLOCK), f32)
        else:
            c = [jnp.sum(ohs[p] * gj, axis=0, keepdims=True)
                 for p in range(k - 1)]
            w = []
            for i in range(k - 1):
                acc = c[i]
                for j in range(i):
                    acc = acc - L[(i + 1, j + 1)] * w[j]
                w.append(acc / L[(i + 1, i + 1)])
            corner = jnp.sqrt(1.0 - sum(wi * wi for wi in w))
            for j in range(k - 1):
                L[(k, j + 1)] = w[j]
            L[(k, k)] = corner

        y = []
        for i in range(k):
            acc = hbs[i]
            for j in range(i):
                acc = acc - L[(i + 1, j + 1)] * y[j]
            y.append(acc / L[(i + 1, i + 1)])
        x = [None] * k
        for i in range(k - 1, -1, -1):
            acc = y[i]
            for j in range(i + 1, k):
                acc = acc - L[(j + 1, i + 1)] * x[j]
            x[i] = acc / L[(i + 1, i + 1)]

        if k == 1:
            h = hbar - x[0] * g0
        elif k < SPARSITY_LEVEL:
            def bt(v):
                return v.astype(jnp.bfloat16).astype(f32)
            beta_v = bt(x[0]) * grows[0]
            for j in range(1, k):
                beta_v = beta_v + bt(x[j]) * grows[j]
            h = hbar - beta_v

    xdense = x[0] * ohs[0]
    for j in range(1, SPARSITY_LEVEL):
        xdense = xdense + x[j] * ohs[j]
    gamma_ref[...] = xdense
    recon = jax.lax.dot_general(
        dn.astype(jnp.bfloat16), xdense.astype(jnp.bfloat16),
        (((1,), (0,)), ((), ())),
        preferred_element_type=jnp.float32)
    recon_ref[...] = recon

    diff = recon - zb
    sse_blk = jnp.sum(diff * diff, keepdims=True).reshape(1, 1)

    mx = jnp.maximum(x[0], 0.0)
    for j in range(1, SPARSITY_LEVEL):
        mx = jnp.maximum(mx, x[j])
    e0 = jnp.exp(-mx)
    es = [jnp.exp(xj - mx) for xj in x]
    denom = (NUM_EMBEDDINGS - SPARSITY_LEVEL) * e0
    for ej in es:
        denom = denom + ej
    p0 = e0 / denom
    ent = (NUM_EMBEDDINGS - SPARSITY_LEVEL) * p0 * jnp.log(p0 + 1e-10)
    for ej in es:
        pj = ej / denom
        ent = ent + pj * jnp.log(pj + 1e-10)
    ent_blk = jnp.sum(ent, keepdims=True).reshape(1, 1)

    first = pl.program_id(0) == 0
    prev_sse = jnp.where(first, jnp.zeros((1, 1), f32), sse_ref[...])
    prev_ent = jnp.where(first, jnp.zeros((1, 1), f32), ent_ref[...])
    sse_ref[...] = prev_sse + sse_blk
    ent_ref[...] = prev_ent + ent_blk


def kernel(z_e, dictionary, beta):
    K, C, B = NUM_EMBEDDINGS, EMBEDDING_DIM, BATCH
    z = jnp.transpose(z_e, (0, 2, 3, 1))
    ze_shape = z.shape
    zf = z.reshape(C, -1)
    dn = dictionary / jnp.linalg.norm(dictionary, axis=0)
    dnt = dn.T

    G = pl.pallas_call(
        _prep_kernel,
        out_shape=jax.ShapeDtypeStruct((K, K), jnp.float32),
    )(dn, dnt)

    nblk = B // BLOCK
    gamma, recon, sse, ent = pl.pallas_call(
        _omp_kernel,
        grid=(nblk,),
        in_specs=[
            pl.BlockSpec((C, BLOCK), lambda i: (0, i)),
            pl.BlockSpec((C, K), lambda i: (0, 0)),
            pl.BlockSpec((K, C), lambda i: (0, 0)),
            pl.BlockSpec((K, K), lambda i: (0, 0)),
        ],
        out_specs=[
            pl.BlockSpec((K, BLOCK), lambda i: (0, i)),
            pl.BlockSpec((C, BLOCK), lambda i: (0, i)),
            pl.BlockSpec((1, 1), lambda i: (0, 0)),
            pl.BlockSpec((1, 1), lambda i: (0, 0)),
        ],
        out_shape=[
            jax.ShapeDtypeStruct((K, B), jnp.float32),
            jax.ShapeDtypeStruct((C, B), jnp.float32),
            jax.ShapeDtypeStruct((1, 1), jnp.float32),
            jax.ShapeDtypeStruct((1, 1), jnp.float32),
        ],
    )(zf, dn, dnt, G)

    mse = sse[0, 0] / (C * B)
    loss = mse * COMMITMENT_COST + beta * mse
    perplexity = jnp.exp(-(ent[0, 0] / B))
    recon_out = jnp.transpose(recon.reshape(ze_shape), (0, 3, 1, 2))
    return loss, recon_out, zf, perplexity, gamma


# speedup vs baseline: 2.9512x; 1.7548x over previous
import jax
import jax.numpy as jnp
from jax.experimental import pallas as pl

NUM_EMBEDDINGS = 1024
EMBEDDING_DIM = 64
COMMITMENT_COST = 0.25
SPARSITY_LEVEL = 4
BATCH = 8192
BLOCK = 256
HI = jax.lax.Precision.HIGHEST


def _prep_kernel(dn_ref, dnt_ref, g1_ref, g2_ref, g3_ref):
    dn = dn_ref[...]
    dnt = dnt_ref[...]
    G = jax.lax.dot_general(
        dnt.astype(jnp.bfloat16), dn.astype(jnp.bfloat16),
        (((1,), (0,)), ((), ())),
        preferred_element_type=jnp.float32)
    g1 = G.astype(jnp.bfloat16)
    r1 = G - g1.astype(jnp.float32)
    g2 = r1.astype(jnp.bfloat16)
    r2 = r1 - g2.astype(jnp.float32)
    g1_ref[...] = g1
    g2_ref[...] = g2
    g3_ref[...] = r2.astype(jnp.bfloat16)


def _omp_kernel(zf_ref, dn_ref, dnt_ref, g1_ref, g2_ref, g3_ref,
                gamma_ref, recon_ref, sse_ref, ent_ref):
    f32 = jnp.float32
    K = NUM_EMBEDDINGS
    zb = zf_ref[...]
    dn = dn_ref[...]
    dnt = dnt_ref[...]
    G1 = g1_ref[...]
    G2 = g2_ref[...]
    G3 = g3_ref[...]

    hbar = jax.lax.dot_general(
        dnt.astype(jnp.bfloat16), zb.astype(jnp.bfloat16),
        (((1,), (0,)), ((), ())), preferred_element_type=jnp.float32)
    h = hbar
    iota = jax.lax.broadcasted_iota(jnp.int32, (K, BLOCK), 0)
    avail = jnp.ones((K, BLOCK), f32)

    ohs = []
    grows = []
    hbs = []
    g0 = None
    L = {}

    for step in range(SPARSITY_LEVEL):
        k = step + 1
        absh = jnp.abs(h) * avail
        m = jnp.max(absh, axis=0, keepdims=True)
        cand = jnp.where(absh >= m, iota, K)
        idx = jnp.min(cand, axis=0, keepdims=True)
        oh = (iota == idx).astype(f32)
        avail = avail * (1.0 - oh)
        ohb = oh.astype(jnp.bfloat16)
        nums = (((1,), (0,)), ((), ()))
        p1 = jax.lax.dot_general(G1, ohb, nums, preferred_element_type=f32)
        p2 = jax.lax.dot_general(G2, ohb, nums, preferred_element_type=f32)
        p3 = jax.lax.dot_general(G3, ohb, nums, preferred_element_type=f32)
        gj = (p1 + p2) + p3
        hbj = jnp.sum(oh * hbar, axis=0, keepdims=True)
        ohs.append(oh)
        hbs.append(hbj)
        if k < SPARSITY_LEVEL:
            if k == 1:
                g0 = gj
            grows.append(p1)

        if k == 1:
            L[(1, 1)] = jnp.ones((1, BLOCK), f32)
        else:
            c = [jnp.sum(ohs[p] * gj, axis=0, keepdims=True)
                 for p in range(k - 1)]
            w = []
            for i in range(k - 1):
                acc = c[i]
                for j in range(i):
                    acc = acc - L[(i + 1, j + 1)] * w[j]
                w.append(acc / L[(i + 1, i + 1)])
            corner = jnp.sqrt(1.0 - sum(wi * wi for wi in w))
            for j in range(k - 1):
                L[(k, j + 1)] = w[j]
            L[(k, k)] = corner

        y = []
        for i in range(k):
            acc = hbs[i]
            for j in range(i):
                acc = acc - L[(i + 1, j + 1)] * y[j]
            y.append(acc / L[(i + 1, i + 1)])
        x = [None] * k
        for i in range(k - 1, -1, -1):
            acc = y[i]
            for j in range(i + 1, k):
                acc = acc - L[(j + 1, i + 1)] * x[j]
            x[i] = acc / L[(i + 1, i + 1)]

        if k == 1:
            h = hbar - x[0] * g0
        elif k < SPARSITY_LEVEL:
            def bt(v):
                return v.astype(jnp.bfloat16).astype(f32)
            beta_v = bt(x[0]) * grows[0]
            for j in range(1, k):
                beta_v = beta_v + bt(x[j]) * grows[j]
            h = hbar - beta_v

    xdense = x[0] * ohs[0]
    for j in range(1, SPARSITY_LEVEL):
        xdense = xdense + x[j] * ohs[j]
    gamma_ref[...] = xdense
    recon = jax.lax.dot_general(
        dn.astype(jnp.bfloat16), xdense.astype(jnp.bfloat16),
        (((1,), (0,)), ((), ())),
        preferred_element_type=jnp.float32)
    recon_ref[...] = recon

    diff = recon - zb
    sse_blk = jnp.sum(diff * diff, keepdims=True).reshape(1, 1)

    mx = jnp.maximum(x[0], 0.0)
    for j in range(1, SPARSITY_LEVEL):
        mx = jnp.maximum(mx, x[j])
    e0 = jnp.exp(-mx)
    es = [jnp.exp(xj - mx) for xj in x]
    denom = (NUM_EMBEDDINGS - SPARSITY_LEVEL) * e0
    for ej in es:
        denom = denom + ej
    p0 = e0 / denom
    ent = (NUM_EMBEDDINGS - SPARSITY_LEVEL) * p0 * jnp.log(p0 + 1e-10)
    for ej in es:
        pj = ej / denom
        ent = ent + pj * jnp.log(pj + 1e-10)
    ent_blk = jnp.sum(ent, keepdims=True).reshape(1, 1)

    first = pl.program_id(0) == 0
    prev_sse = jnp.where(first, jnp.zeros((1, 1), f32), sse_ref[...])
    prev_ent = jnp.where(first, jnp.zeros((1, 1), f32), ent_ref[...])
    sse_ref[...] = prev_sse + sse_blk
    ent_ref[...] = prev_ent + ent_blk


def kernel(z_e, dictionary, beta):
    K, C, B = NUM_EMBEDDINGS, EMBEDDING_DIM, BATCH
    z = jnp.transpose(z_e, (0, 2, 3, 1))
    ze_shape = z.shape
    zf = z.reshape(C, -1)
    dn = dictionary / jnp.linalg.norm(dictionary, axis=0)
    dnt = dn.T

    G1, G2, G3 = pl.pallas_call(
        _prep_kernel,
        out_shape=[
            jax.ShapeDtypeStruct((K, K), jnp.bfloat16),
            jax.ShapeDtypeStruct((K, K), jnp.bfloat16),
            jax.ShapeDtypeStruct((K, K), jnp.bfloat16),
        ],
    )(dn, dnt)

    nblk = B // BLOCK
    gamma, recon, sse, ent = pl.pallas_call(
        _omp_kernel,
        grid=(nblk,),
        in_specs=[
            pl.BlockSpec((C, BLOCK), lambda i: (0, i)),
            pl.BlockSpec((C, K), lambda i: (0, 0)),
            pl.BlockSpec((K, C), lambda i: (0, 0)),
            pl.BlockSpec((K, K), lambda i: (0, 0)),
            pl.BlockSpec((K, K), lambda i: (0, 0)),
            pl.BlockSpec((K, K), lambda i: (0, 0)),
        ],
        out_specs=[
            pl.BlockSpec((K, BLOCK), lambda i: (0, i)),
            pl.BlockSpec((C, BLOCK), lambda i: (0, i)),
            pl.BlockSpec((1, 1), lambda i: (0, 0)),
            pl.BlockSpec((1, 1), lambda i: (0, 0)),
        ],
        out_shape=[
            jax.ShapeDtypeStruct((K, B), jnp.float32),
            jax.ShapeDtypeStruct((C, B), jnp.float32),
            jax.ShapeDtypeStruct((1, 1), jnp.float32),
            jax.ShapeDtypeStruct((1, 1), jnp.float32),
        ],
    )(zf, dn, dnt, G1, G2, G3)

    mse = sse[0, 0] / (C * B)
    loss = mse * COMMITMENT_COST + beta * mse
    perplexity = jnp.exp(-(ent[0, 0] / B))
    recon_out = jnp.transpose(recon.reshape(ze_shape), (0, 3, 1, 2))
    return loss, recon_out, zf, perplexity, gamma


# speedup vs baseline: 4.7092x; 1.5957x over previous
import jax
import jax.numpy as jnp
from jax.experimental import pallas as pl

NUM_EMBEDDINGS = 1024
EMBEDDING_DIM = 64
COMMITMENT_COST = 0.25
SPARSITY_LEVEL = 4
BATCH = 8192
BLOCK = 512
HI = jax.lax.Precision.HIGHEST


def _prep_kernel(dn_ref, dnt_ref, g1_ref, g2_ref, g3_ref):
    dn = dn_ref[...]
    dnt = dnt_ref[...]
    G = jax.lax.dot_general(
        dnt.astype(jnp.bfloat16), dn.astype(jnp.bfloat16),
        (((1,), (0,)), ((), ())),
        preferred_element_type=jnp.float32)
    g1 = G.astype(jnp.bfloat16)
    r1 = G - g1.astype(jnp.float32)
    g2 = r1.astype(jnp.bfloat16)
    r2 = r1 - g2.astype(jnp.float32)
    g1_ref[...] = g1
    g2_ref[...] = g2
    g3_ref[...] = r2.astype(jnp.bfloat16)


def _omp_kernel(zf_ref, dn_ref, dnt_ref, g1_ref, g2_ref, g3_ref,
                gamma_ref, recon_ref, sse_ref, ent_ref):
    f32 = jnp.float32
    K = NUM_EMBEDDINGS
    zb = zf_ref[...]
    dn = dn_ref[...]
    dnt = dnt_ref[...]
    G1 = g1_ref[...]
    G2 = g2_ref[...]
    G3 = g3_ref[...]

    hbar = jax.lax.dot_general(
        dnt.astype(jnp.bfloat16), zb.astype(jnp.bfloat16),
        (((1,), (0,)), ((), ())), preferred_element_type=jnp.float32)
    h = hbar
    iota = jax.lax.broadcasted_iota(jnp.int32, (K, BLOCK), 0)
    avail = jnp.ones((K, BLOCK), f32)

    ohs = []
    grows = []
    hbs = []
    g0 = None
    L = {}

    for step in range(SPARSITY_LEVEL):
        k = step + 1
        absh = jnp.abs(h) * avail
        m = jnp.max(absh, axis=0, keepdims=True)
        cand = jnp.where(absh >= m, iota, K)
        idx = jnp.min(cand, axis=0, keepdims=True)
        oh = (iota == idx).astype(f32)
        avail = avail * (1.0 - oh)
        ohb = oh.astype(jnp.bfloat16)
        nums = (((1,), (0,)), ((), ()))
        p1 = jax.lax.dot_general(G1, ohb, nums, preferred_element_type=f32)
        p2 = jax.lax.dot_general(G2, ohb, nums, preferred_element_type=f32)
        p3 = jax.lax.dot_general(G3, ohb, nums, preferred_element_type=f32)
        gj = (p1 + p2) + p3
        hbj = jnp.sum(oh * hbar, axis=0, keepdims=True)
        ohs.append(oh)
        hbs.append(hbj)
        if k < SPARSITY_LEVEL:
            if k == 1:
                g0 = gj
            grows.append(p1)

        if k == 1:
            L[(1, 1)] = jnp.ones((1, BLOCK), f32)
        else:
            c = [jnp.sum(ohs[p] * gj, axis=0, keepdims=True)
                 for p in range(k - 1)]
            w = []
            for i in range(k - 1):
                acc = c[i]
                for j in range(i):
                    acc = acc - L[(i + 1, j + 1)] * w[j]
                w.append(acc / L[(i + 1, i + 1)])
            corner = jnp.sqrt(1.0 - sum(wi * wi for wi in w))
            for j in range(k - 1):
                L[(k, j + 1)] = w[j]
            L[(k, k)] = corner

        y = []
        for i in range(k):
            acc = hbs[i]
            for j in range(i):
                acc = acc - L[(i + 1, j + 1)] * y[j]
            y.append(acc / L[(i + 1, i + 1)])
        x = [None] * k
        for i in range(k - 1, -1, -1):
            acc = y[i]
            for j in range(i + 1, k):
                acc = acc - L[(j + 1, i + 1)] * x[j]
            x[i] = acc / L[(i + 1, i + 1)]

        if k == 1:
            h = hbar - x[0] * g0
        elif k < SPARSITY_LEVEL:
            def bt(v):
                return v.astype(jnp.bfloat16).astype(f32)
            beta_v = bt(x[0]) * grows[0]
            for j in range(1, k):
                beta_v = beta_v + bt(x[j]) * grows[j]
            h = hbar - beta_v

    xdense = x[0] * ohs[0]
    for j in range(1, SPARSITY_LEVEL):
        xdense = xdense + x[j] * ohs[j]
    gamma_ref[...] = xdense
    recon = jax.lax.dot_general(
        dn.astype(jnp.bfloat16), xdense.astype(jnp.bfloat16),
        (((1,), (0,)), ((), ())),
        preferred_element_type=jnp.float32)
    recon_ref[...] = recon

    diff = recon - zb
    sse_blk = jnp.sum(diff * diff, keepdims=True).reshape(1, 1)

    mx = jnp.maximum(x[0], 0.0)
    for j in range(1, SPARSITY_LEVEL):
        mx = jnp.maximum(mx, x[j])
    e0 = jnp.exp(-mx)
    es = [jnp.exp(xj - mx) for xj in x]
    denom = (NUM_EMBEDDINGS - SPARSITY_LEVEL) * e0
    for ej in es:
        denom = denom + ej
    p0 = e0 / denom
    ent = (NUM_EMBEDDINGS - SPARSITY_LEVEL) * p0 * jnp.log(p0 + 1e-10)
    for ej in es:
        pj = ej / denom
        ent = ent + pj * jnp.log(pj + 1e-10)
    ent_blk = jnp.sum(ent, keepdims=True).reshape(1, 1)

    first = pl.program_id(0) == 0
    prev_sse = jnp.where(first, jnp.zeros((1, 1), f32), sse_ref[...])
    prev_ent = jnp.where(first, jnp.zeros((1, 1), f32), ent_ref[...])
    sse_ref[...] = prev_sse + sse_blk
    ent_ref[...] = prev_ent + ent_blk


def kernel(z_e, dictionary, beta):
    K, C, B = NUM_EMBEDDINGS, EMBEDDING_DIM, BATCH
    z = jnp.transpose(z_e, (0, 2, 3, 1))
    ze_shape = z.shape
    zf = z.reshape(C, -1)
    dn = dictionary / jnp.linalg.norm(dictionary, axis=0)
    dnt = dn.T

    G1, G2, G3 = pl.pallas_call(
        _prep_kernel,
        out_shape=[
            jax.ShapeDtypeStruct((K, K), jnp.bfloat16),
            jax.ShapeDtypeStruct((K, K), jnp.bfloat16),
            jax.ShapeDtypeStruct((K, K), jnp.bfloat16),
        ],
    )(dn, dnt)

    nblk = B // BLOCK
    gamma, recon, sse, ent = pl.pallas_call(
        _omp_kernel,
        grid=(nblk,),
        in_specs=[
            pl.BlockSpec((C, BLOCK), lambda i: (0, i)),
            pl.BlockSpec((C, K), lambda i: (0, 0)),
            pl.BlockSpec((K, C), lambda i: (0, 0)),
            pl.BlockSpec((K, K), lambda i: (0, 0)),
            pl.BlockSpec((K, K), lambda i: (0, 0)),
            pl.BlockSpec((K, K), lambda i: (0, 0)),
        ],
        out_specs=[
            pl.BlockSpec((K, BLOCK), lambda i: (0, i)),
            pl.BlockSpec((C, BLOCK), lambda i: (0, i)),
            pl.BlockSpec((1, 1), lambda i: (0, 0)),
            pl.BlockSpec((1, 1), lambda i: (0, 0)),
        ],
        out_shape=[
            jax.ShapeDtypeStruct((K, B), jnp.float32),
            jax.ShapeDtypeStruct((C, B), jnp.float32),
            jax.ShapeDtypeStruct((1, 1), jnp.float32),
            jax.ShapeDtypeStruct((1, 1), jnp.float32),
        ],
    )(zf, dn, dnt, G1, G2, G3)

    mse = sse[0, 0] / (C * B)
    loss = mse * COMMITMENT_COST + beta * mse
    perplexity = jnp.exp(-(ent[0, 0] / B))
    recon_out = jnp.transpose(recon.reshape(ze_shape), (0, 3, 1, 2))
    return loss, recon_out, zf, perplexity, gamma
